# Initial kernel scaffold; baseline (speedup 1.0000x reference)
import functools

import jax
import jax.numpy as jnp
from jax import lax
from jax.experimental import pallas as pl
from jax.experimental.pallas import tpu as pltpu
from jax.experimental.pallas import tpu_sc as plsc

SEQ = 200
BATCH = 16384
VOCAB = 1000000
NC = 2
NS = 16
NW = NC * NS
BPW = BATCH // NW
NCHUNK = BPW // 128
LANES = 16


def _lr_body(inp_hbm, lut_hbm, bias_hbm, out_hbm,
             table_sh, idx_v, vals_v, acc_v, bias_sm, sem_g):
    cid = lax.axis_index("c")
    sid = lax.axis_index("s")
    wid = cid * NS + sid

    @pl.when(sid == 0)
    def _():
        pltpu.sync_copy(lut_hbm, table_sh)

    pltpu.sync_copy(bias_hbm, bias_sm)
    plsc.subcore_barrier()

    bvec = jnp.full((LANES,), bias_sm[0], dtype=jnp.float32)
    for i in range(BPW // LANES):
        acc_v[pl.ds(i * LANES, LANES)] = bvec

    def step(s, carry):
        pltpu.sync_copy(inp_hbm.at[s, wid], idx_v)
        copies = [
            pltpu.async_copy(table_sh.at[idx_v.at[j]], vals_v.at[j], sem_g)
            for j in range(NCHUNK)
        ]
        for c in copies:
            c.wait()
        for j in range(NCHUNK):
            for i in range(128 // LANES):
                v = vals_v[j, pl.ds(i * LANES, LANES)]
                plsc.addupdate(acc_v.at[pl.ds(j * 128 + i * LANES, LANES)], v)
        return carry

    lax.fori_loop(0, SEQ, step, 0)

    pltpu.sync_copy(acc_v, out_hbm.at[pl.ds(wid * BPW, BPW)])


@jax.jit
def kernel(input, lut, bias):
    inp = input.reshape(SEQ, NW, NCHUNK, 128)
    lut_flat = lut.reshape(VOCAB)
    run = pl.kernel(
        _lr_body,
        out_type=jax.ShapeDtypeStruct((BATCH,), jnp.float32),
        mesh=plsc.VectorSubcoreMesh(core_axis_name="c", subcore_axis_name="s"),
        scratch_types=[
            pltpu.VMEM_SHARED((VOCAB,), jnp.float32),
            pltpu.VMEM((NCHUNK, 128), jnp.int32),
            pltpu.VMEM((NCHUNK, 128), jnp.float32),
            pltpu.VMEM((BPW,), jnp.float32),
            pltpu.SMEM((1,), jnp.float32),
            pltpu.SemaphoreType.DMA,
        ],
    )
    return run(inp, lut_flat, bias)


# speedup vs baseline: 121.8940x; 121.8940x over previous
import functools

import jax
import jax.numpy as jnp
from jax import lax
from jax.experimental import pallas as pl
from jax.experimental.pallas import tpu as pltpu
from jax.experimental.pallas import tpu_sc as plsc

SEQ = 200
BATCH = 16384
VOCAB = 1000000
NC = 2
NS = 16
NW = NC * NS
BPW = BATCH // NW
NCHUNK = BPW // 128
LANES = 16


def _lr_body(inp_hbm, lut_hbm, bias_hbm, out_hbm,
             table_sh, idx_v, vals_v, acc_v, bias_sm, sem_g):
    cid = lax.axis_index("c")
    sid = lax.axis_index("s")
    wid = cid * NS + sid

    @pl.when(sid == 0)
    def _():
        pltpu.sync_copy(lut_hbm, table_sh)

    bias_sm[...] = jnp.zeros((LANES,), jnp.float32)
    pltpu.sync_copy(bias_hbm, bias_sm.at[pl.ds(0, 1)])
    plsc.subcore_barrier()

    zero_idx = jnp.zeros((LANES,), jnp.int32)
    bvec = bias_sm[...].at[zero_idx].get(mode="promise_in_bounds")
    for i in range(BPW // LANES):
        acc_v[pl.ds(i * LANES, LANES)] = bvec

    def step(s, carry):
        pltpu.sync_copy(inp_hbm.at[s, wid], idx_v)
        copies = [
            pltpu.async_copy(table_sh.at[idx_v.at[j]], vals_v.at[j], sem_g)
            for j in range(NCHUNK)
        ]
        for c in copies:
            c.wait()
        for j in range(NCHUNK):
            for i in range(128 // LANES):
                v = vals_v[j, pl.ds(i * LANES, LANES)]
                plsc.addupdate(acc_v.at[pl.ds(j * 128 + i * LANES, LANES)], v)
        return carry

    lax.fori_loop(0, SEQ, step, 0)

    pltpu.sync_copy(acc_v, out_hbm.at[pl.ds(wid * BPW, BPW)])


@jax.jit
def kernel(input, lut, bias):
    inp = input.reshape(SEQ, NW, NCHUNK, 128)
    lut_flat = lut.reshape(VOCAB)
    run = pl.kernel(
        _lr_body,
        out_type=jax.ShapeDtypeStruct((BATCH,), jnp.float32),
        mesh=plsc.VectorSubcoreMesh(core_axis_name="c", subcore_axis_name="s"),
        scratch_types=[
            pltpu.VMEM_SHARED((VOCAB,), jnp.float32),
            pltpu.VMEM((NCHUNK, 128), jnp.int32),
            pltpu.VMEM((NCHUNK, 128), jnp.float32),
            pltpu.VMEM((BPW,), jnp.float32),
            pltpu.VMEM((LANES,), jnp.float32),
            pltpu.SemaphoreType.DMA,
        ],
    )
    return run(inp, lut_flat, bias)


# speedup vs baseline: 211.3004x; 1.7335x over previous
import functools

import jax
import jax.numpy as jnp
from jax import lax
from jax.experimental import pallas as pl
from jax.experimental.pallas import tpu as pltpu
from jax.experimental.pallas import tpu_sc as plsc

SEQ = 200
BATCH = 16384
VOCAB = 1000000
NC = 2
NS = 16
NW = NC * NS
BPW = BATCH // NW
NCHUNK = BPW // 128
LANES = 16


NIBUF = 4
NVBUF = 2


def _lr_body(inp_hbm, lut_hbm, bias_hbm, out_hbm,
             table_sh, idx_v, vals_v, acc_v, bias_sm,
             isem0, isem1, isem2, isem3, gsem0, gsem1):
    cid = lax.axis_index("c")
    sid = lax.axis_index("s")
    wid = cid * NS + sid
    isems = (isem0, isem1, isem2, isem3)
    gsems = (gsem0, gsem1)

    @pl.when(sid == 0)
    def _():
        pltpu.sync_copy(lut_hbm, table_sh)

    bias_sm[...] = jnp.zeros((LANES,), jnp.float32)
    pltpu.sync_copy(bias_hbm, bias_sm.at[pl.ds(0, 1)])
    plsc.subcore_barrier()

    zero_idx = jnp.zeros((LANES,), jnp.int32)
    bvec = bias_sm[...].at[zero_idx].get(mode="promise_in_bounds")
    for i in range(BPW // LANES):
        acc_v[pl.ds(i * LANES, LANES)] = bvec

    def idx_desc(s, islot):
        return pltpu.make_async_copy(
            inp_hbm.at[s, wid], idx_v.at[islot], isems[islot])

    def gather_descs(s, vslot, islot):
        del s
        return [
            pltpu.make_async_copy(
                table_sh.at[idx_v.at[islot, j]], vals_v.at[vslot, j],
                gsems[vslot])
            for j in range(NCHUNK)
        ]

    def accum(vslot):
        for j in range(NCHUNK):
            for i in range(128 // LANES):
                v = vals_v[vslot, j, pl.ds(i * LANES, LANES)]
                plsc.addupdate(acc_v.at[pl.ds(j * 128 + i * LANES, LANES)], v)

    idx_desc(0, 0).start()
    idx_desc(1, 1).start()
    idx_desc(0, 0).wait()
    for d in gather_descs(0, 0, 0):
        d.start()

    def body(t, carry):
        s0 = t * NIBUF
        for k in range(NIBUF):
            s = s0 + k
            idx_desc(s + 2, (k + 2) % NIBUF).start()
            idx_desc(s + 1, (k + 1) % NIBUF).wait()
            for d in gather_descs(s + 1, (k + 1) % NVBUF, (k + 1) % NIBUF):
                d.start()
            for d in gather_descs(s, k % NVBUF, k % NIBUF):
                d.wait()
            accum(k % NVBUF)
        return carry

    lax.fori_loop(0, (SEQ - 4) // NIBUF, body, 0)

    for k in range(NIBUF):
        s = SEQ - 4 + k
        if k + 2 < NIBUF:
            idx_desc(s + 2, (k + 2) % NIBUF).start()
        if k + 1 < NIBUF:
            idx_desc(s + 1, (k + 1) % NIBUF).wait()
            for d in gather_descs(s + 1, (k + 1) % NVBUF, (k + 1) % NIBUF):
                d.start()
        for d in gather_descs(s, k % NVBUF, k % NIBUF):
            d.wait()
        accum(k % NVBUF)

    pltpu.sync_copy(acc_v, out_hbm.at[pl.ds(wid * BPW, BPW)])


@jax.jit
def kernel(input, lut, bias):
    inp = input.reshape(SEQ, NW, NCHUNK, 128)
    lut_flat = lut.reshape(VOCAB)
    run = pl.kernel(
        _lr_body,
        out_type=jax.ShapeDtypeStruct((BATCH,), jnp.float32),
        mesh=plsc.VectorSubcoreMesh(core_axis_name="c", subcore_axis_name="s"),
        scratch_types=[
            pltpu.VMEM_SHARED((VOCAB,), jnp.float32),
            pltpu.VMEM((NIBUF, NCHUNK, 128), jnp.int32),
            pltpu.VMEM((NVBUF, NCHUNK, 128), jnp.float32),
            pltpu.VMEM((BPW,), jnp.float32),
            pltpu.VMEM((LANES,), jnp.float32),
            pltpu.SemaphoreType.DMA,
            pltpu.SemaphoreType.DMA,
            pltpu.SemaphoreType.DMA,
            pltpu.SemaphoreType.DMA,
            pltpu.SemaphoreType.DMA,
            pltpu.SemaphoreType.DMA,
        ],
    )
    return run(inp, lut_flat, bias)


# speedup vs baseline: 236.5211x; 1.1194x over previous
import functools

import jax
import jax.numpy as jnp
from jax import lax
from jax.experimental import pallas as pl
from jax.experimental.pallas import tpu as pltpu
from jax.experimental.pallas import tpu_sc as plsc

SEQ = 200
BATCH = 16384
VOCAB = 1000000
NC = 2
NS = 16
NW = NC * NS
BPW = BATCH // NW
NCHUNK = BPW // 128
LANES = 16


NIBUF = 4
NVBUF = 2


def _lr_body(inp_hbm, lut_hbm, bias_hbm, out_hbm,
             table_sh, idx_v, vals_v, acc_v, bias_sm,
             isem0, isem1, isem2, isem3, gsem0, gsem1):
    cid = lax.axis_index("c")
    sid = lax.axis_index("s")
    wid = cid * NS + sid
    isems = (isem0, isem1, isem2, isem3)
    gsems = (gsem0, gsem1)

    @pl.when(sid == 0)
    def _():
        pltpu.sync_copy(lut_hbm, table_sh)

    bias_sm[...] = jnp.zeros((LANES,), jnp.float32)
    pltpu.sync_copy(bias_hbm, bias_sm.at[pl.ds(0, 1)])
    plsc.subcore_barrier()

    zero_idx = jnp.zeros((LANES,), jnp.int32)
    bvec = bias_sm[...].at[zero_idx].get(mode="promise_in_bounds")
    for i in range(BPW // LANES):
        acc_v[pl.ds(i * LANES, LANES)] = bvec

    def idx_desc(s, islot):
        return pltpu.make_async_copy(
            inp_hbm.at[s, pl.ds(wid * BPW, BPW)], idx_v.at[islot],
            isems[islot])

    def gather_descs(s, vslot, islot):
        del s
        return [
            pltpu.make_async_copy(
                table_sh.at[idx_v.at[islot, pl.ds(j * 128, 128)]],
                vals_v.at[vslot, j], gsems[vslot])
            for j in range(NCHUNK)
        ]

    def accum(vslot):
        for j in range(NCHUNK):
            for i in range(128 // LANES):
                v = vals_v[vslot, j, pl.ds(i * LANES, LANES)]
                plsc.addupdate(acc_v.at[pl.ds(j * 128 + i * LANES, LANES)], v)

    idx_desc(0, 0).start()
    idx_desc(1, 1).start()
    idx_desc(0, 0).wait()
    for d in gather_descs(0, 0, 0):
        d.start()

    def body(t, carry):
        s0 = t * NIBUF
        for k in range(NIBUF):
            s = s0 + k
            idx_desc(s + 2, (k + 2) % NIBUF).start()
            idx_desc(s + 1, (k + 1) % NIBUF).wait()
            for d in gather_descs(s + 1, (k + 1) % NVBUF, (k + 1) % NIBUF):
                d.start()
            for d in gather_descs(s, k % NVBUF, k % NIBUF):
                d.wait()
            accum(k % NVBUF)
        return carry

    lax.fori_loop(0, (SEQ - 4) // NIBUF, body, 0)

    for k in range(NIBUF):
        s = SEQ - 4 + k
        if k + 2 < NIBUF:
            idx_desc(s + 2, (k + 2) % NIBUF).start()
        if k + 1 < NIBUF:
            idx_desc(s + 1, (k + 1) % NIBUF).wait()
            for d in gather_descs(s + 1, (k + 1) % NVBUF, (k + 1) % NIBUF):
                d.start()
        for d in gather_descs(s, k % NVBUF, k % NIBUF):
            d.wait()
        accum(k % NVBUF)

    pltpu.sync_copy(acc_v, out_hbm.at[pl.ds(wid * BPW, BPW)])


@jax.jit
def kernel(input, lut, bias):
    inp = input
    lut_flat = lut.reshape(VOCAB)
    run = pl.kernel(
        _lr_body,
        out_type=jax.ShapeDtypeStruct((BATCH,), jnp.float32),
        mesh=plsc.VectorSubcoreMesh(core_axis_name="c", subcore_axis_name="s"),
        scratch_types=[
            pltpu.VMEM_SHARED((VOCAB,), jnp.float32),
            pltpu.VMEM((NIBUF, BPW), jnp.int32),
            pltpu.VMEM((NVBUF, NCHUNK, 128), jnp.float32),
            pltpu.VMEM((BPW,), jnp.float32),
            pltpu.VMEM((LANES,), jnp.float32),
            pltpu.SemaphoreType.DMA,
            pltpu.SemaphoreType.DMA,
            pltpu.SemaphoreType.DMA,
            pltpu.SemaphoreType.DMA,
            pltpu.SemaphoreType.DMA,
            pltpu.SemaphoreType.DMA,
        ],
    )
    return run(inp, lut_flat, bias)
